# baseline (speedup 1.0000x reference)
import functools

import jax
import jax.numpy as jnp
from jax import lax
from jax.experimental import pallas as pl
from jax.experimental.pallas import tpu as pltpu
from jax.experimental.pallas import tpu_sc as plsc

_DIM = 4096
_NE = 64
_TOPK = 8
_T = 16384
_BT = 512
_HB = _BT // 2
_HBS = _HB.bit_length() - 1
_CHUNKS = (8192, 4096, 4096)

_NC = 2
_NS = 16
_NW = _NC * _NS


def _matmul_body(x_ref, w_ref, o_ref):
    s = lax.dot_general(
        x_ref[...], w_ref[...],
        dimension_numbers=(((1,), (1,)), ((), ())),
        preferred_element_type=jnp.float32,
    )
    o_ref[...] = jnp.concatenate([s[:_HB], s[_HB:]], axis=1)


def _scores_tc(x, weight, rows, row0):
    blk0 = row0 // _BT
    return pl.pallas_call(
        _matmul_body,
        grid=(rows // _BT,),
        in_specs=[
            pl.BlockSpec((_BT, _DIM), lambda i: (blk0 + i, 0)),
            pl.BlockSpec((_NE, _DIM), lambda i: (0, 0)),
        ],
        out_specs=pl.BlockSpec((_HB, 2 * _NE), lambda i: (i, 0)),
        out_shape=jax.ShapeDtypeStruct((rows // 2, 2 * _NE), jnp.float32),
    )(x, weight)


_NPASS = 2


@functools.cache
def _topk_sc(rows):
    pr = rows // 2 // _NW
    assert _HB % pr == 0 or pr % _HB == 0
    pp = pr // _NPASS
    n_active = _NW

    def _topk_body(scores_hbm, wout_hbm, iout_hbm, sbuf, wbuf, ibuf):
        wid = lax.axis_index("s") * _NC + lax.axis_index("c")
        base = wid * pr
        tok0 = (wid * pr // _HB) * _BT + (wid * pr) % _HB

        lanes = lax.iota(jnp.int32, 16)
        in_lo = lanes < 8

        def _merge(ka, va, kb, vb):
            kb_r = lax.rev(kb, (0,))
            vb_r = lax.rev(vb, (0,))
            k = jnp.where(in_lo, ka, kb_r)
            v = jnp.where(in_lo, va, vb_r)
            return plsc.sort_key_val(k, v, descending=True)

        def _top8(p, lane0):
            srt = []
            for j in range(4):
                k = sbuf[p, pl.ds(lane0 + 16 * j, 16)]
                srt.append(
                    plsc.sort_key_val(k, lanes + 16 * j, descending=True))
            k01, v01 = _merge(*srt[0], *srt[1])
            k23, v23 = _merge(*srt[2], *srt[3])
            kf, vf = _merge(k01, v01, k23, v23)
            m = jnp.max(kf)
            e = jnp.exp(kf - m)
            e = jnp.where(in_lo, e, 0.0)
            s = jnp.broadcast_to(jnp.sum(e), (16,))
            return e / s, vf

        for ps in range(_NPASS):
            pltpu.sync_copy(
                scores_hbm.at[pl.ds(base + ps * pp, pp)], sbuf)

            @plsc.parallel_loop(0, pp, step=1, unroll=8)
            def _rows(p):
                for lane0, rt in ((0, p), (64, pp + p)):
                    w, v = _top8(p, lane0)
                    rowv = jnp.broadcast_to(rt, (16,))
                    plsc.store_scatter(wbuf, [rowv, lanes], w, mask=in_lo)
                    plsc.store_scatter(ibuf, [rowv, lanes], v, mask=in_lo)

            a0 = tok0 + ps * pp
            b0 = a0 + _HB
            pltpu.sync_copy(wbuf.at[pl.ds(0, pp)],
                            wout_hbm.at[pl.ds(a0, pp)])
            pltpu.sync_copy(wbuf.at[pl.ds(pp, pp)],
                            wout_hbm.at[pl.ds(b0, pp)])
            pltpu.sync_copy(ibuf.at[pl.ds(0, pp)],
                            iout_hbm.at[pl.ds(a0, pp)])
            pltpu.sync_copy(ibuf.at[pl.ds(pp, pp)],
                            iout_hbm.at[pl.ds(b0, pp)])

    return pl.kernel(
        _topk_body,
        out_type=(
            jax.ShapeDtypeStruct((rows, _TOPK), jnp.float32),
            jax.ShapeDtypeStruct((rows, _TOPK), jnp.int32),
        ),
        mesh=plsc.VectorSubcoreMesh(core_axis_name="c", subcore_axis_name="s"),
        compiler_params=pltpu.CompilerParams(
            needs_layout_passes=False, use_tc_tiling_on_sc=True),
        scratch_types=[
            pltpu.VMEM((pr // _NPASS, 2 * _NE), jnp.float32),
            pltpu.VMEM((2 * pr // _NPASS, _TOPK), jnp.float32),
            pltpu.VMEM((2 * pr // _NPASS, _TOPK), jnp.int32),
        ],
    )


_BTA = 2048


def _asm_body(wp_ref, ip_ref, w_ref, i_ref):
    w_ref[...] = wp_ref[...].reshape(_BTA, _TOPK)
    i_ref[...] = ip_ref[...].reshape(_BTA, _TOPK)


def _assemble_tc(wp, ip, rows):
    return pl.pallas_call(
        _asm_body,
        grid=(rows // _BTA,),
        in_specs=[
            pl.BlockSpec((_BTA // 16, 128), lambda i: (i, 0)),
            pl.BlockSpec((_BTA // 16, 128), lambda i: (i, 0)),
        ],
        out_specs=[
            pl.BlockSpec((_BTA, _TOPK), lambda i: (i, 0)),
            pl.BlockSpec((_BTA, _TOPK), lambda i: (i, 0)),
        ],
        out_shape=(
            jax.ShapeDtypeStruct((rows, _TOPK), jnp.float32),
            jax.ShapeDtypeStruct((rows, _TOPK), jnp.int32),
        ),
    )(wp, ip)


def kernel(x, weight):
    outs = []
    row0 = 0
    for crows in _CHUNKS:
        scores = _scores_tc(x, weight, crows, row0)
        outs.append(_topk_sc(crows)(scores))
        row0 += crows
    w8 = jnp.concatenate([o[0] for o in outs], axis=0)
    i8 = jnp.concatenate([o[1] for o in outs], axis=0)
    return (w8, i8)


# speedup vs baseline: 1.0570x; 1.0570x over previous
import functools

import jax
import jax.numpy as jnp
from jax import lax
from jax.experimental import pallas as pl
from jax.experimental.pallas import tpu as pltpu
from jax.experimental.pallas import tpu_sc as plsc

_DIM = 4096
_NE = 64
_TOPK = 8
_T = 16384
_BT = 512
_HB = _BT // 2
_HBS = _HB.bit_length() - 1
_CHUNKS = (8192, 8192)

_NC = 2
_NS = 16
_NW = _NC * _NS


def _matmul_body(x_ref, w_ref, o_ref):
    s = lax.dot_general(
        x_ref[...], w_ref[...],
        dimension_numbers=(((1,), (1,)), ((), ())),
        preferred_element_type=jnp.float32,
    )
    o_ref[...] = jnp.concatenate([s[:_HB], s[_HB:]], axis=1)


def _scores_tc(x, weight, rows, row0):
    blk0 = row0 // _BT
    return pl.pallas_call(
        _matmul_body,
        grid=(rows // _BT,),
        in_specs=[
            pl.BlockSpec((_BT, _DIM), lambda i: (blk0 + i, 0)),
            pl.BlockSpec((_NE, _DIM), lambda i: (0, 0)),
        ],
        out_specs=pl.BlockSpec((_HB, 2 * _NE), lambda i: (i, 0)),
        out_shape=jax.ShapeDtypeStruct((rows // 2, 2 * _NE), jnp.float32),
    )(x, weight)


@functools.cache
def _topk_sc(rows):
    pr = rows // 2 // _NW
    assert _HB % pr == 0 or pr % _HB == 0
    npass = 1 if pr <= 128 else 2
    pp = pr // npass
    n_active = _NW

    def _topk_body(scores_hbm, wout_hbm, iout_hbm, sbuf, wbuf, ibuf, sem):
        wid = lax.axis_index("s") * _NC + lax.axis_index("c")
        base = wid * pr
        tok0 = (wid * pr // _HB) * _BT + (wid * pr) % _HB

        lanes = lax.iota(jnp.int32, 16)
        in_lo = lanes < 8

        def _merge(ka, va, kb, vb):
            kb_r = lax.rev(kb, (0,))
            vb_r = lax.rev(vb, (0,))
            k = jnp.where(in_lo, ka, kb_r)
            v = jnp.where(in_lo, va, vb_r)
            return plsc.sort_key_val(k, v, descending=True)

        def _top8(p, lane0):
            srt = []
            for j in range(4):
                k = sbuf[p, pl.ds(lane0 + 16 * j, 16)]
                srt.append(
                    plsc.sort_key_val(k, lanes + 16 * j, descending=True))
            k01, v01 = _merge(*srt[0], *srt[1])
            k23, v23 = _merge(*srt[2], *srt[3])
            kf, vf = _merge(k01, v01, k23, v23)
            m = jnp.max(kf)
            e = jnp.exp(kf - m)
            e = jnp.where(in_lo, e, 0.0)
            s = jnp.broadcast_to(jnp.sum(e), (16,))
            return e / s, vf

        for ps in range(npass):
            pltpu.sync_copy(
                scores_hbm.at[pl.ds(base + ps * pp, pp)], sbuf)

            @plsc.parallel_loop(0, pp, step=1, unroll=8)
            def _rows(p):
                for lane0, rt in ((0, p), (64, pp + p)):
                    w, v = _top8(p, lane0)
                    rowv = jnp.broadcast_to(rt, (16,))
                    plsc.store_scatter(wbuf, [rowv, lanes], w, mask=in_lo)
                    plsc.store_scatter(ibuf, [rowv, lanes], v, mask=in_lo)

            a0 = tok0 + ps * pp
            b0 = a0 + _HB
            hs = [
                pltpu.async_copy(wbuf.at[pl.ds(0, pp)],
                                 wout_hbm.at[pl.ds(a0, pp)], sem),
                pltpu.async_copy(wbuf.at[pl.ds(pp, pp)],
                                 wout_hbm.at[pl.ds(b0, pp)], sem),
                pltpu.async_copy(ibuf.at[pl.ds(0, pp)],
                                 iout_hbm.at[pl.ds(a0, pp)], sem),
                pltpu.async_copy(ibuf.at[pl.ds(pp, pp)],
                                 iout_hbm.at[pl.ds(b0, pp)], sem),
            ]
            for h in hs:
                h.wait()

    return pl.kernel(
        _topk_body,
        out_type=(
            jax.ShapeDtypeStruct((rows, _TOPK), jnp.float32),
            jax.ShapeDtypeStruct((rows, _TOPK), jnp.int32),
        ),
        mesh=plsc.VectorSubcoreMesh(core_axis_name="c", subcore_axis_name="s"),
        compiler_params=pltpu.CompilerParams(
            needs_layout_passes=False, use_tc_tiling_on_sc=True),
        scratch_types=[
            pltpu.VMEM((pr // npass, 2 * _NE), jnp.float32),
            pltpu.VMEM((2 * pr // npass, _TOPK), jnp.float32),
            pltpu.VMEM((2 * pr // npass, _TOPK), jnp.int32),
            pltpu.SemaphoreType.DMA,
        ],
    )


_BTA = 2048


def _asm_body(wp_ref, ip_ref, w_ref, i_ref):
    w_ref[...] = wp_ref[...].reshape(_BTA, _TOPK)
    i_ref[...] = ip_ref[...].reshape(_BTA, _TOPK)


def _assemble_tc(wp, ip, rows):
    return pl.pallas_call(
        _asm_body,
        grid=(rows // _BTA,),
        in_specs=[
            pl.BlockSpec((_BTA // 16, 128), lambda i: (i, 0)),
            pl.BlockSpec((_BTA // 16, 128), lambda i: (i, 0)),
        ],
        out_specs=[
            pl.BlockSpec((_BTA, _TOPK), lambda i: (i, 0)),
            pl.BlockSpec((_BTA, _TOPK), lambda i: (i, 0)),
        ],
        out_shape=(
            jax.ShapeDtypeStruct((rows, _TOPK), jnp.float32),
            jax.ShapeDtypeStruct((rows, _TOPK), jnp.int32),
        ),
    )(wp, ip)


def kernel(x, weight):
    outs = []
    row0 = 0
    for crows in _CHUNKS:
        scores = _scores_tc(x, weight, crows, row0)
        outs.append(_topk_sc(crows)(scores))
        row0 += crows
    w8 = jnp.concatenate([o[0] for o in outs], axis=0)
    i8 = jnp.concatenate([o[1] for o in outs], axis=0)
    return (w8, i8)
